# Initial kernel scaffold; baseline (speedup 1.0000x reference)
import jax
import jax.numpy as jnp
from jax.experimental import pallas as pl


def kernel(ind_2, p1, p3, basis, d3, Wpre0, bpre0, Wpre1, bpre1, Wpi, bpi, Wii0, Wii1, Wpo0, Wpo1, Weq, Wpp0, bpp0, Wpp1, bpp1):
    raise NotImplementedError("write your pallas kernel here")


# speedup vs baseline: 8.8812x; 8.8812x over previous
import functools
import jax
import jax.numpy as jnp
from jax import lax
from jax.experimental import pallas as pl
from jax.experimental.pallas import tpu as pltpu
from jax.experimental.pallas import tpu_sc as plsc

N = 10000
P = 160000
C = 64
NB = 10

NC = 2
NS = 16
NW = NC * NS
CHUNK = 128
P2 = 163840
NCH = P2 // (NW * CHUNK)
PPW = P2 // NW
NACC = 10240
STRIPE = NACC // NS
f32 = jnp.float32
_HI = jax.lax.Precision.HIGHEST

_mesh = plsc.VectorSubcoreMesh(core_axis_name="c", subcore_axis_name="s",
                               num_cores=NC, num_subcores=NS)


def _tanh(x):
    t = jnp.exp(-2.0 * jnp.abs(x))
    return jnp.sign(x) * (1.0 - t) / (1.0 + t)


def _tc_h_body(p1_ref, w0_ref, b0_ref, w1_ref, b1_ref, h_ref):
    h = _tanh(jnp.dot(p1_ref[...], w0_ref[...],
                      preferred_element_type=f32) + b0_ref[...])
    h_ref[...] = _tanh(jnp.dot(h, w1_ref[...],
                               preferred_element_type=f32) + b1_ref[...])


def _tc_h(p1_pad, W0, b0, W1, b1):
    return pl.pallas_call(
        _tc_h_body,
        out_shape=jax.ShapeDtypeStruct((NACC, C), f32),
    )(p1_pad, W0, b0, W1, b1)


def _sc_gather_body(h_hbm, i2d_hbm, j2d_hbm, hi_hbm, hj_hbm,
                    idxi_v, idxj_v, rowsi_v, rowsj_v, semi, semj):
    w = lax.axis_index("c") * NS + lax.axis_index("s")
    irow = w * NCH
    pltpu.sync_copy(i2d_hbm.at[pl.ds(irow, NCH)], idxi_v)
    pltpu.sync_copy(j2d_hbm.at[pl.ds(irow, NCH)], idxj_v)
    base = w * PPW

    def body(c, carry):
        cpi = pltpu.async_copy(h_hbm.at[idxi_v.at[c]], rowsi_v, semi)
        cpj = pltpu.async_copy(h_hbm.at[idxj_v.at[c]], rowsj_v, semj)
        cpi.wait()
        pltpu.sync_copy(rowsi_v, hi_hbm.at[pl.ds(base + c * CHUNK, CHUNK)])
        cpj.wait()
        pltpu.sync_copy(rowsj_v, hj_hbm.at[pl.ds(base + c * CHUNK, CHUNK)])
        return carry

    lax.fori_loop(0, NCH, body, 0)


@functools.partial(
    pl.kernel, mesh=_mesh,
    compiler_params=pltpu.CompilerParams(use_tc_tiling_on_sc=False),
    out_type=[jax.ShapeDtypeStruct((P2, C), f32),
              jax.ShapeDtypeStruct((P2, C), f32)],
    scratch_types=[pltpu.VMEM((NCH, CHUNK), jnp.int32),
                   pltpu.VMEM((NCH, CHUNK), jnp.int32),
                   pltpu.VMEM((CHUNK, C), f32),
                   pltpu.VMEM((CHUNK, C), f32),
                   pltpu.SemaphoreType.DMA,
                   pltpu.SemaphoreType.DMA])
def _sc_gather(*args):
    _sc_gather_body(*args)


def _tc_pair_body(hi_ref, hj_ref, bas_ref, wa_ref, wb_ref, bpi_ref,
                  h_ref, s_ref, wii0_ref, wii1_ref, out_ref):
    t = _tanh(jnp.dot(hi_ref[...], wa_ref[...], preferred_element_type=f32)
              + jnp.dot(hj_ref[...], wb_ref[...], preferred_element_type=f32)
              + bpi_ref[...])
    u = t * jnp.dot(bas_ref[...], h_ref[...],
                    preferred_element_type=f32, precision=_HI)
    ip = jnp.dot(u, s_ref[...], preferred_element_type=f32, precision=_HI)
    s1 = _tanh(jnp.dot(ip, wii0_ref[...], preferred_element_type=f32))
    out_ref[...] = _tanh(jnp.dot(s1, wii1_ref[...],
                                 preferred_element_type=f32))


def _tc_pair(hi, hj, basis_pad, WpiA, WpiB, bpi2, Hmat, Smat, Wii0, Wii1):
    B = 1024
    grid = (P2 // B,)
    wspec = lambda shape: pl.BlockSpec(shape, lambda n: (0, 0))
    return pl.pallas_call(
        _tc_pair_body,
        grid=grid,
        in_specs=[
            pl.BlockSpec((B, C), lambda n: (n, 0)),
            pl.BlockSpec((B, C), lambda n: (n, 0)),
            pl.BlockSpec((B, NB), lambda n: (n, 0)),
            wspec((C, C * NB)),
            wspec((C, C * NB)),
            wspec((1, C * NB)),
            wspec((NB, C * NB)),
            wspec((C * NB, C)),
            wspec((C, C)),
            wspec((C, 2 * C)),
        ],
        out_specs=pl.BlockSpec((B, 2 * C), lambda n: (n, 0)),
        out_shape=jax.ShapeDtypeStruct((P2, 2 * C), f32),
    )(hi, hj, basis_pad, WpiA, WpiB, bpi2, Hmat, Smat, Wii0, Wii1)


def _zero_buf(zbuf, ncols):
    zero16 = jnp.zeros((16,), f32)

    def zr(r, carry):
        for k in range(ncols // 16):
            zbuf[r, pl.ds(k * 16, 16)] = zero16
        return carry

    lax.fori_loop(0, CHUNK, zr, 0)


def _sc_scatter_p1_body(ipair_hbm, i2d_hbm, out_hbm, idx_v, rows_v, acc):
    core = lax.axis_index("c")
    s = lax.axis_index("s")
    w = core * NS + s

    _zero_buf(rows_v, 2 * C)
    for q in range(STRIPE // CHUNK):
        pltpu.sync_copy(rows_v, acc.at[pl.ds(s * STRIPE + q * CHUNK, CHUNK)])
    plsc.subcore_barrier()

    pltpu.sync_copy(i2d_hbm.at[pl.ds(w * NCH, NCH)], idx_v)
    base = w * PPW

    def body(c, carry):
        pltpu.sync_copy(ipair_hbm.at[pl.ds(base + c * CHUNK, CHUNK)], rows_v)
        pltpu.sync_copy(rows_v, acc.at[idx_v.at[c]], add=True)
        return carry

    lax.fori_loop(0, NCH, body, 0)
    plsc.subcore_barrier()

    for q in range(STRIPE // CHUNK):
        r = s * STRIPE + q * CHUNK
        pltpu.sync_copy(acc.at[pl.ds(r, CHUNK)], rows_v)
        pltpu.sync_copy(rows_v, out_hbm.at[core, pl.ds(r, CHUNK)])


@functools.partial(
    pl.kernel, mesh=_mesh,
    compiler_params=pltpu.CompilerParams(use_tc_tiling_on_sc=False),
    out_type=jax.ShapeDtypeStruct((NC, NACC, 2 * C), f32),
    scratch_types=[pltpu.VMEM((NCH, CHUNK), jnp.int32),
                   pltpu.VMEM((CHUNK, 2 * C), f32),
                   pltpu.VMEM_SHARED((NACC, 2 * C), f32)])
def _sc_scatter_p1(*args):
    _sc_scatter_p1_body(*args)


def _sc_equivar_body(p3x0_hbm, p3x1_hbm, p3x2_hbm, d3x0_hbm, d3x1_hbm,
                     d3x2_hbm, ipair_hbm, i2d_hbm, j2d_hbm, out_hbm,
                     idxi_v, idxj_v, g_v, b_v, dx_v, acc, sem):
    core = lax.axis_index("c")
    s = lax.axis_index("s")
    w = core * NS + s
    base = w * PPW

    pltpu.sync_copy(i2d_hbm.at[pl.ds(w * NCH, NCH)], idxi_v)
    pltpu.sync_copy(j2d_hbm.at[pl.ds(w * NCH, NCH)], idxj_v)

    p3xs = [p3x0_hbm, p3x1_hbm, p3x2_hbm]
    d3xs = [d3x0_hbm, d3x1_hbm, d3x2_hbm]

    for x in range(3):
        _zero_buf(g_v, C)
        for q in range(STRIPE // CHUNK):
            pltpu.sync_copy(g_v, acc.at[pl.ds(s * STRIPE + q * CHUNK, CHUNK)])
        plsc.subcore_barrier()

        def body(c, carry):
            pltpu.async_copy(p3xs[x].at[idxj_v.at[c]], g_v, sem).wait()
            pltpu.sync_copy(
                ipair_hbm.at[pl.ds(base + c * CHUNK, CHUNK), pl.ds(C, C)],
                b_v)
            pltpu.sync_copy(d3xs[x].at[pl.ds(base + c * CHUNK, CHUNK)], dx_v)

            def grp(pg, carry2):
                dvec = dx_v[pl.ds(pg * 16, 16)]
                for lane in range(16):
                    p = pg * 16 + lane
                    dx = dvec[lane]
                    for cb in range(4):
                        o = cb * 16
                        g_v[p, pl.ds(o, 16)] = (
                            (g_v[p, pl.ds(o, 16)] + dx) * b_v[p, pl.ds(o, 16)])
                return carry2

            lax.fori_loop(0, CHUNK // 16, grp, 0)
            pltpu.sync_copy(g_v, acc.at[idxi_v.at[c]], add=True)
            return carry

        lax.fori_loop(0, NCH, body, 0)
        plsc.subcore_barrier()

        for q in range(STRIPE // CHUNK):
            r = s * STRIPE + q * CHUNK
            pltpu.sync_copy(acc.at[pl.ds(r, CHUNK)], g_v)
            pltpu.sync_copy(g_v, out_hbm.at[x, core, pl.ds(r, CHUNK)])
        plsc.subcore_barrier()


@functools.partial(
    pl.kernel, mesh=_mesh,
    compiler_params=pltpu.CompilerParams(use_tc_tiling_on_sc=False),
    out_type=jax.ShapeDtypeStruct((3, NC, NACC, C), f32),
    scratch_types=[pltpu.VMEM((NCH, CHUNK), jnp.int32),
                   pltpu.VMEM((NCH, CHUNK), jnp.int32),
                   pltpu.VMEM((CHUNK, C), f32),
                   pltpu.VMEM((CHUNK, C), f32),
                   pltpu.VMEM((CHUNK,), f32),
                   pltpu.VMEM_SHARED((NACC, C), f32),
                   pltpu.SemaphoreType.DMA])
def _sc_equivar(*args):
    _sc_equivar_body(*args)


def _tc_final_body(p1s_ref, p3s_ref, wpo0_ref, wpo1_ref, weq_ref,
                   wa_ref, wb_ref, bpp0_ref, wpp1_ref, bpp1_ref,
                   p1o_ref, p3o_ref):
    p1s = p1s_ref[0] + p1s_ref[1]
    p1n = _tanh(jnp.dot(p1s, wpo0_ref[...], preferred_element_type=f32))
    p1n = _tanh(jnp.dot(p1n, wpo1_ref[...], preferred_element_type=f32))
    p3n = [jnp.dot(p3s_ref[x, 0] + p3s_ref[x, 1], weq_ref[...],
                   preferred_element_type=f32) for x in range(3)]
    dotted = p3n[0] * p3n[0] + p3n[1] * p3n[1] + p3n[2] * p3n[2]
    z = _tanh(jnp.dot(p1n, wa_ref[...], preferred_element_type=f32)
              + jnp.dot(dotted, wb_ref[...], preferred_element_type=f32)
              + bpp0_ref[...])
    z = _tanh(jnp.dot(z, wpp1_ref[...], preferred_element_type=f32)
              + bpp1_ref[...])
    p1o_ref[...] = z[:, :C]
    sc = z[:, C:]
    for x in range(3):
        p3o_ref[:, pl.ds(x * C, C)] = p3n[x] * sc


def _tc_final(p1s, p3s, Wpo0, Wpo1, Weq, Wpp0A, Wpp0B, bpp02, Wpp1, bpp12):
    B = 1000
    grid = (N // B,)
    wspec = lambda shape: pl.BlockSpec(shape, lambda n: tuple(0 for _ in shape))
    return pl.pallas_call(
        _tc_final_body,
        grid=grid,
        in_specs=[
            pl.BlockSpec((NC, B, 2 * C), lambda n: (0, n, 0)),
            pl.BlockSpec((3, NC, B, C), lambda n: (0, 0, n, 0)),
            wspec((2 * C, C)),
            wspec((C, C)),
            wspec((C, C)),
            wspec((C, C)),
            wspec((C, C)),
            wspec((1, C)),
            wspec((C, 2 * C)),
            wspec((1, 2 * C)),
        ],
        out_specs=[pl.BlockSpec((B, C), lambda n: (n, 0)),
                   pl.BlockSpec((B, 3 * C), lambda n: (n, 0))],
        out_shape=[jax.ShapeDtypeStruct((N, C), f32),
                   jax.ShapeDtypeStruct((N, 3 * C), f32)],
    )(p1s, p3s, Wpo0, Wpo1, Weq, Wpp0A, Wpp0B, bpp02, Wpp1, bpp12)


def kernel(ind_2, p1, p3, basis, d3, Wpre0, bpre0, Wpre1, bpre1, Wpi, bpi,
           Wii0, Wii1, Wpo0, Wpo1, Weq, Wpp0, bpp0, Wpp1, bpp1):
    pad = P2 - P
    i_pad = jnp.concatenate(
        [ind_2[:, 0], N + (jnp.arange(pad, dtype=jnp.int32) % CHUNK)])
    j_pad = jnp.concatenate(
        [ind_2[:, 1], jnp.zeros((pad,), jnp.int32)])
    i2d = i_pad.reshape(-1, CHUNK)
    j2d = j_pad.reshape(-1, CHUNK)
    basis_pad = jnp.concatenate([basis, jnp.zeros((pad, NB), f32)])
    zpad = jnp.zeros((pad,), f32)
    d3x = [jnp.concatenate([d3[:, x], zpad]) for x in range(3)]
    p3x = [p3[:, x, :] for x in range(3)]
    p1_pad = jnp.concatenate([p1, jnp.zeros((NACC - N, C), f32)])

    WpiA, WpiB = Wpi[:C], Wpi[C:]
    bpi2 = bpi.reshape(1, -1)
    Hmat = jnp.tile(jnp.eye(NB, dtype=f32), (1, C))
    Smat = jnp.repeat(jnp.eye(C, dtype=f32), NB, axis=0)
    Wpp0A, Wpp0B = Wpp0[:C], Wpp0[C:]
    bpp02 = bpp0.reshape(1, -1)
    bpp12 = bpp1.reshape(1, -1)

    h = _tc_h(p1_pad, Wpre0, bpre0.reshape(1, -1), Wpre1, bpre1.reshape(1, -1))
    hi, hj = _sc_gather(h, i2d, j2d)
    ipair = _tc_pair(hi, hj, basis_pad, WpiA, WpiB, bpi2, Hmat, Smat, Wii0, Wii1)
    p1s = _sc_scatter_p1(ipair, i2d)
    p3s = _sc_equivar(p3x[0], p3x[1], p3x[2], d3x[0], d3x[1], d3x[2],
                      ipair, i2d, j2d)
    p1_out, p3flat_out = _tc_final(p1s, p3s, Wpo0, Wpo1, Weq,
                                   Wpp0A, Wpp0B, bpp02, Wpp1, bpp12)
    return p1_out, p3flat_out.reshape(N, 3, C)


# speedup vs baseline: 9.3796x; 1.0561x over previous
import functools
import jax
import jax.numpy as jnp
from jax import lax
from jax.experimental import pallas as pl
from jax.experimental.pallas import tpu as pltpu
from jax.experimental.pallas import tpu_sc as plsc

N = 10000
P = 160000
C = 64
NB = 10

NC = 2
NS = 16
NW = NC * NS
CHUNK = 128
P2 = 163840
NCH = P2 // (NW * CHUNK)
PPW = P2 // NW
NACC = 10240
STRIPE = NACC // NS
f32 = jnp.float32
_HI = jax.lax.Precision.HIGHEST

_mesh = plsc.VectorSubcoreMesh(core_axis_name="c", subcore_axis_name="s",
                               num_cores=NC, num_subcores=NS)


def _tanh(x):
    t = jnp.exp(-2.0 * jnp.abs(x))
    return jnp.sign(x) * (1.0 - t) / (1.0 + t)


def _tc_h_body(p1_ref, w0_ref, b0_ref, w1_ref, b1_ref, h_ref):
    h = _tanh(jnp.dot(p1_ref[...], w0_ref[...],
                      preferred_element_type=f32) + b0_ref[...])
    h_ref[...] = _tanh(jnp.dot(h, w1_ref[...],
                               preferred_element_type=f32) + b1_ref[...])


def _tc_h(p1_pad, W0, b0, W1, b1):
    return pl.pallas_call(
        _tc_h_body,
        out_shape=jax.ShapeDtypeStruct((NACC, C), f32),
    )(p1_pad, W0, b0, W1, b1)


def _sc_gather_body(h_hbm, i2d_hbm, j2d_hbm, hi_hbm, hj_hbm,
                    idxi_v, idxj_v, rowsi_v, rowsj_v, semi, semj):
    w = lax.axis_index("c") * NS + lax.axis_index("s")
    irow = w * NCH
    pltpu.sync_copy(i2d_hbm.at[pl.ds(irow, NCH)], idxi_v)
    pltpu.sync_copy(j2d_hbm.at[pl.ds(irow, NCH)], idxj_v)
    base = w * PPW

    def body(c, carry):
        cpi = pltpu.async_copy(h_hbm.at[idxi_v.at[c]], rowsi_v, semi)
        cpj = pltpu.async_copy(h_hbm.at[idxj_v.at[c]], rowsj_v, semj)
        cpi.wait()
        pltpu.sync_copy(rowsi_v, hi_hbm.at[pl.ds(base + c * CHUNK, CHUNK)])
        cpj.wait()
        pltpu.sync_copy(rowsj_v, hj_hbm.at[pl.ds(base + c * CHUNK, CHUNK)])
        return carry

    lax.fori_loop(0, NCH, body, 0)


@functools.partial(
    pl.kernel, mesh=_mesh,
    compiler_params=pltpu.CompilerParams(use_tc_tiling_on_sc=False),
    out_type=[jax.ShapeDtypeStruct((P2, C), f32),
              jax.ShapeDtypeStruct((P2, C), f32)],
    scratch_types=[pltpu.VMEM((NCH, CHUNK), jnp.int32),
                   pltpu.VMEM((NCH, CHUNK), jnp.int32),
                   pltpu.VMEM((CHUNK, C), f32),
                   pltpu.VMEM((CHUNK, C), f32),
                   pltpu.SemaphoreType.DMA,
                   pltpu.SemaphoreType.DMA])
def _sc_gather(*args):
    _sc_gather_body(*args)


def _tc_pair_body(hi_ref, hj_ref, bas_ref, wa_ref, wb_ref, bpi_ref,
                  h_ref, s_ref, wii0_ref, wii1_ref, out_ref):
    t = _tanh(jnp.dot(hi_ref[...], wa_ref[...], preferred_element_type=f32)
              + jnp.dot(hj_ref[...], wb_ref[...], preferred_element_type=f32)
              + bpi_ref[...])
    u = t * jnp.dot(bas_ref[...], h_ref[...],
                    preferred_element_type=f32, precision=_HI)
    ip = jnp.dot(u, s_ref[...], preferred_element_type=f32, precision=_HI)
    s1 = _tanh(jnp.dot(ip, wii0_ref[...], preferred_element_type=f32))
    out_ref[...] = _tanh(jnp.dot(s1, wii1_ref[...],
                                 preferred_element_type=f32))


def _tc_pair(hi, hj, basis_pad, WpiA, WpiB, bpi2, Hmat, Smat, Wii0, Wii1):
    B = 1024
    grid = (P2 // B,)
    wspec = lambda shape: pl.BlockSpec(shape, lambda n: (0, 0))
    return pl.pallas_call(
        _tc_pair_body,
        grid=grid,
        in_specs=[
            pl.BlockSpec((B, C), lambda n: (n, 0)),
            pl.BlockSpec((B, C), lambda n: (n, 0)),
            pl.BlockSpec((B, NB), lambda n: (n, 0)),
            wspec((C, C * NB)),
            wspec((C, C * NB)),
            wspec((1, C * NB)),
            wspec((NB, C * NB)),
            wspec((C * NB, C)),
            wspec((C, C)),
            wspec((C, 2 * C)),
        ],
        out_specs=pl.BlockSpec((B, 2 * C), lambda n: (n, 0)),
        out_shape=jax.ShapeDtypeStruct((P2, 2 * C), f32),
    )(hi, hj, basis_pad, WpiA, WpiB, bpi2, Hmat, Smat, Wii0, Wii1)


def _zero_buf(zbuf, ncols):
    zero16 = jnp.zeros((16,), f32)

    def zr(r, carry):
        for k in range(ncols // 16):
            zbuf[r, pl.ds(k * 16, 16)] = zero16
        return carry

    lax.fori_loop(0, CHUNK, zr, 0)


def _sc_scatter_p1_body(ipair_hbm, i2d_hbm, out_hbm, idx_v, rows_v, acc):
    core = lax.axis_index("c")
    s = lax.axis_index("s")
    w = core * NS + s

    _zero_buf(rows_v, 2 * C)
    for q in range(STRIPE // CHUNK):
        pltpu.sync_copy(rows_v, acc.at[pl.ds(s * STRIPE + q * CHUNK, CHUNK)])
    plsc.subcore_barrier()

    pltpu.sync_copy(i2d_hbm.at[pl.ds(w * NCH, NCH)], idx_v)
    base = w * PPW

    def body(c, carry):
        pltpu.sync_copy(ipair_hbm.at[pl.ds(base + c * CHUNK, CHUNK)], rows_v)
        pltpu.sync_copy(rows_v, acc.at[idx_v.at[c]], add=True)
        return carry

    lax.fori_loop(0, NCH, body, 0)
    plsc.subcore_barrier()

    for q in range(STRIPE // CHUNK):
        r = s * STRIPE + q * CHUNK
        pltpu.sync_copy(acc.at[pl.ds(r, CHUNK)], rows_v)
        pltpu.sync_copy(rows_v, out_hbm.at[core, pl.ds(r, CHUNK)])


@functools.partial(
    pl.kernel, mesh=_mesh,
    compiler_params=pltpu.CompilerParams(use_tc_tiling_on_sc=False),
    out_type=jax.ShapeDtypeStruct((NC, NACC, 2 * C), f32),
    scratch_types=[pltpu.VMEM((NCH, CHUNK), jnp.int32),
                   pltpu.VMEM((CHUNK, 2 * C), f32),
                   pltpu.VMEM_SHARED((NACC, 2 * C), f32)])
def _sc_scatter_p1(*args):
    _sc_scatter_p1_body(*args)


def _sc_equivar_body(p3x0_hbm, p3x1_hbm, p3x2_hbm, d3x0_hbm, d3x1_hbm,
                     d3x2_hbm, ipair_hbm, i2d_hbm, j2d_hbm, out_hbm,
                     idxi_v, idxj_v, g_v, b_v, dx_v, acc, sem):
    core = lax.axis_index("c")
    s = lax.axis_index("s")
    w = core * NS + s
    base = w * PPW

    pltpu.sync_copy(i2d_hbm.at[pl.ds(w * NCH, NCH)], idxi_v)
    pltpu.sync_copy(j2d_hbm.at[pl.ds(w * NCH, NCH)], idxj_v)

    p3s = [p3x0_hbm, p3x1_hbm, p3x2_hbm]
    d3s = [d3x0_hbm, d3x1_hbm, d3x2_hbm]

    for x in range(3):
        _zero_buf(g_v, C)
        for q in range(STRIPE // CHUNK):
            pltpu.sync_copy(g_v, acc.at[pl.ds(s * STRIPE + q * CHUNK, CHUNK)])
        plsc.subcore_barrier()

        def body(c, carry):
            cp = pltpu.async_copy(p3s[x].at[idxj_v.at[c]], g_v, sem)
            pltpu.sync_copy(
                ipair_hbm.at[pl.ds(base + c * CHUNK, CHUNK), pl.ds(C, C)], b_v)
            pltpu.sync_copy(d3s[x].at[pl.ds(base + c * CHUNK, CHUNK)], dx_v)
            cp.wait()

            def grp(pg, carry2):
                dvec = dx_v[pl.ds(pg * 16, 16)]
                for lane in range(16):
                    p = pg * 16 + lane
                    dx = dvec[lane]
                    for cb in range(4):
                        o = cb * 16
                        g_v[p, pl.ds(o, 16)] = (
                            (g_v[p, pl.ds(o, 16)] + dx)
                            * b_v[p, pl.ds(o, 16)])
                return carry2

            lax.fori_loop(0, CHUNK // 16, grp, 0)
            pltpu.sync_copy(g_v, acc.at[idxi_v.at[c]], add=True)
            return carry

        lax.fori_loop(0, NCH, body, 0)
        plsc.subcore_barrier()

        for q in range(STRIPE // CHUNK):
            r = s * STRIPE + q * CHUNK
            pltpu.sync_copy(acc.at[pl.ds(r, CHUNK)], g_v)
            pltpu.sync_copy(g_v, out_hbm.at[core, x, pl.ds(r, CHUNK)])


@functools.partial(
    pl.kernel, mesh=_mesh,
    compiler_params=pltpu.CompilerParams(use_tc_tiling_on_sc=False),
    out_type=jax.ShapeDtypeStruct((NC, 3, NACC, C), f32),
    scratch_types=[pltpu.VMEM((NCH, CHUNK), jnp.int32),
                   pltpu.VMEM((NCH, CHUNK), jnp.int32),
                   pltpu.VMEM((CHUNK, C), f32),
                   pltpu.VMEM((CHUNK, C), f32),
                   pltpu.VMEM((CHUNK,), f32),
                   pltpu.VMEM_SHARED((NACC, C), f32),
                   pltpu.SemaphoreType.DMA])
def _sc_equivar(*args):
    _sc_equivar_body(*args)


def _tc_final_body(p1s_ref, p3s_ref, wpo0_ref, wpo1_ref, weq_ref,
                   wa_ref, wb_ref, bpp0_ref, wpp1_ref, bpp1_ref,
                   p1o_ref, p3o_ref):
    p1s = p1s_ref[0] + p1s_ref[1]
    p1n = _tanh(jnp.dot(p1s, wpo0_ref[...], preferred_element_type=f32))
    p1n = _tanh(jnp.dot(p1n, wpo1_ref[...], preferred_element_type=f32))
    p3n = [jnp.dot(p3s_ref[0, x] + p3s_ref[1, x], weq_ref[...],
                   preferred_element_type=f32) for x in range(3)]
    dotted = p3n[0] * p3n[0] + p3n[1] * p3n[1] + p3n[2] * p3n[2]
    z = _tanh(jnp.dot(p1n, wa_ref[...], preferred_element_type=f32)
              + jnp.dot(dotted, wb_ref[...], preferred_element_type=f32)
              + bpp0_ref[...])
    z = _tanh(jnp.dot(z, wpp1_ref[...], preferred_element_type=f32)
              + bpp1_ref[...])
    p1o_ref[...] = z[:, :C]
    sc = z[:, C:]
    for x in range(3):
        p3o_ref[:, pl.ds(x * C, C)] = p3n[x] * sc


def _tc_final(p1s, p3s, Wpo0, Wpo1, Weq, Wpp0A, Wpp0B, bpp02, Wpp1, bpp12):
    B = 1000
    grid = (N // B,)
    wspec = lambda shape: pl.BlockSpec(shape, lambda n: tuple(0 for _ in shape))
    return pl.pallas_call(
        _tc_final_body,
        grid=grid,
        in_specs=[
            pl.BlockSpec((NC, B, 2 * C), lambda n: (0, n, 0)),
            pl.BlockSpec((NC, 3, B, C), lambda n: (0, 0, n, 0)),
            wspec((2 * C, C)),
            wspec((C, C)),
            wspec((C, C)),
            wspec((C, C)),
            wspec((C, C)),
            wspec((1, C)),
            wspec((C, 2 * C)),
            wspec((1, 2 * C)),
        ],
        out_specs=[pl.BlockSpec((B, C), lambda n: (n, 0)),
                   pl.BlockSpec((B, 3 * C), lambda n: (n, 0))],
        out_shape=[jax.ShapeDtypeStruct((N, C), f32),
                   jax.ShapeDtypeStruct((N, 3 * C), f32)],
    )(p1s, p3s, Wpo0, Wpo1, Weq, Wpp0A, Wpp0B, bpp02, Wpp1, bpp12)


def kernel(ind_2, p1, p3, basis, d3, Wpre0, bpre0, Wpre1, bpre1, Wpi, bpi,
           Wii0, Wii1, Wpo0, Wpo1, Weq, Wpp0, bpp0, Wpp1, bpp1):
    pad = P2 - P
    i_pad = jnp.concatenate(
        [ind_2[:, 0], N + (jnp.arange(pad, dtype=jnp.int32) % CHUNK)])
    j_pad = jnp.concatenate(
        [ind_2[:, 1], jnp.zeros((pad,), jnp.int32)])
    i2d = i_pad.reshape(-1, CHUNK)
    j2d = j_pad.reshape(-1, CHUNK)
    basis_pad = jnp.concatenate([basis, jnp.zeros((pad, NB), f32)])
    zpad = jnp.zeros((pad,), f32)
    d3x = [jnp.concatenate([d3[:, x], zpad]) for x in range(3)]
    p3x = [p3[:, x, :] for x in range(3)]
    p1_pad = jnp.concatenate([p1, jnp.zeros((NACC - N, C), f32)])

    WpiA, WpiB = Wpi[:C], Wpi[C:]
    bpi2 = bpi.reshape(1, -1)
    Hmat = jnp.tile(jnp.eye(NB, dtype=f32), (1, C))
    Smat = jnp.repeat(jnp.eye(C, dtype=f32), NB, axis=0)
    Wpp0A, Wpp0B = Wpp0[:C], Wpp0[C:]
    bpp02 = bpp0.reshape(1, -1)
    bpp12 = bpp1.reshape(1, -1)

    h = _tc_h(p1_pad, Wpre0, bpre0.reshape(1, -1), Wpre1, bpre1.reshape(1, -1))
    hi, hj = _sc_gather(h, i2d, j2d)
    ipair = _tc_pair(hi, hj, basis_pad, WpiA, WpiB, bpi2, Hmat, Smat, Wii0, Wii1)
    p1s = _sc_scatter_p1(ipair, i2d)
    p3s = _sc_equivar(p3x[0], p3x[1], p3x[2], d3x[0], d3x[1], d3x[2],
                      ipair, i2d, j2d)
    p1_out, p3flat_out = _tc_final(p1s, p3s, Wpo0, Wpo1, Weq,
                                   Wpp0A, Wpp0B, bpp02, Wpp1, bpp12)
    return p1_out, p3flat_out.reshape(N, 3, C)
